# baseline (speedup 1.0000x reference)
import functools
import jax
import jax.numpy as jnp
from jax import lax
from jax.experimental import pallas as pl
from jax.experimental.pallas import tpu as pltpu
from jax.experimental.pallas import tpu_sc as plsc

N = 10000
E = 320000
D = 128
G = 16

NC = 2
NS = 16
L = 16
NW = NC * NS

NPAD = 10240
NROWS = NPAD // L
RPT = NROWS // NS
RPW = NROWS // NW
EP = E // NW
EITER = EP // L
UNROLL = 25

_f32 = jnp.float32
_i32 = jnp.int32


def _enc_body(x_ref, w1_ref, b1_ref, wa_ref, h_ref, as_ref, ad_ref, m_ref):
    i = pl.program_id(0)
    R = x_ref.shape[0]
    h = x_ref[...] @ w1_ref[...] + b1_ref[...]
    h = jnp.where(h > 0, h, jnp.exp(jnp.minimum(h, 0.0)) - 1.0)
    rows = lax.broadcasted_iota(_i32, (R, 1), 0) + i * R
    h = jnp.where(rows < N, h, 0.0)
    h_ref[...] = h
    a = lax.dot_general(wa_ref[...], h, (((0,), (1,)), ((), ())),
                        preferred_element_type=_f32)
    as_ref[pl.ds(i * R, R)] = a[0]
    ad_ref[pl.ds(i * R, R)] = a[1]
    bm = jnp.max(a, axis=1, keepdims=True) + jnp.zeros((8, 128), _f32)

    @pl.when(i == 0)
    def _():
        m_ref[...] = bm

    @pl.when(i > 0)
    def _():
        m_ref[...] = jnp.maximum(m_ref[...], bm)


def _encode(x, w1, b1_2d, wa8):
    R = 1024
    grid = NPAD // R
    return pl.pallas_call(
        _enc_body,
        grid=(grid,),
        in_specs=[
            pl.BlockSpec((R, D), lambda i: (i, 0)),
            pl.BlockSpec((D, D), lambda i: (0, 0)),
            pl.BlockSpec((1, D), lambda i: (0, 0)),
            pl.BlockSpec((D, 8), lambda i: (0, 0)),
        ],
        out_specs=[
            pl.BlockSpec((R, D), lambda i: (i, 0)),
            pl.BlockSpec((NPAD,), lambda i: (0,)),
            pl.BlockSpec((NPAD,), lambda i: (0,)),
            pl.BlockSpec((8, 128), lambda i: (0, 0)),
        ],
        out_shape=[
            jax.ShapeDtypeStruct((NPAD, D), _f32),
            jax.ShapeDtypeStruct((NPAD,), _f32),
            jax.ShapeDtypeStruct((NPAD,), _f32),
            jax.ShapeDtypeStruct((8, 128), _f32),
        ],
    )(x, w1, b1_2d, wa8)


def _row_iota(ref):
    it = lax.iota(_i32, L)

    def body(r, c):
        ref[pl.ds(r * L, L)] = it + r * L
        return c

    lax.fori_loop(0, NROWS // L, body, 0)


def _combine_and_emit(loc, shared, idxr, chunk, out_h, cid, sid):
    pltpu.sync_copy(loc, shared.at[idxr], add=True)
    plsc.subcore_barrier()
    pltpu.sync_copy(shared.at[pl.ds(sid * RPT, RPT)], chunk)
    pltpu.sync_copy(chunk, out_h.at[cid, pl.ds(sid * RPT, RPT)])


EPC = E // NS
DITER = EPC // L


def _edge_fused_body(ei_h, asrc_h, adst_h, m_h, z_h, cpart_h,
                     asrc_v, adst_v, src_v, dst_v, m_v, loc, idxr, chunk,
                     sem, dtmp, rinv, shared_d, shared_c):
    cid = lax.axis_index("c")
    sid = lax.axis_index("s")
    base = sid * EPC
    cps = [
        pltpu.async_copy(asrc_h, asrc_v, sem),
        pltpu.async_copy(adst_h, adst_v, sem),
        pltpu.async_copy(ei_h.at[0, pl.ds(base, EPC)], src_v, sem),
        pltpu.async_copy(ei_h.at[1, pl.ds(base, EPC)], dst_v, sem),
        pltpu.async_copy(m_h, m_v, sem),
        pltpu.async_copy(z_h, loc, sem),
    ]

    @pl.when(sid == 0)
    def _():
        pltpu.async_copy(z_h, shared_d, sem).wait()
        pltpu.async_copy(z_h, shared_c, sem).wait()

    _row_iota(idxr)
    for cp in cps:
        cp.wait()
    plsc.subcore_barrier()
    m = m_v[...]

    def dstep(i, c):
        for u in range(UNROLL):
            o = (i * UNROLL + u) * L
            sv = src_v[pl.ds(o, L)]
            dv = dst_v[pl.ds(o, L)]
            a1 = plsc.load_gather(asrc_v, [sv])
            a2 = plsc.load_gather(adst_v, [dv])
            ex = jnp.exp(a1 + a2 - m)
            rv = lax.shift_right_logical(dv, 4)
            cv = lax.bitwise_and(dv, 15)
            plsc.addupdate_scatter(loc, [rv, cv], ex)
        return c

    lax.fori_loop(0, DITER // UNROLL, dstep, 0)

    rbd = sid * RPT

    def sdstep(rr, c):
        r = rbd + rr

        @pl.when(r < N // L)
        def _():
            ex = jnp.exp(asrc_v[pl.ds(r * L, L)]
                         + adst_v[pl.ds(r * L, L)] - m)
            loc[r] = loc[r] + ex

        return c

    lax.fori_loop(0, RPT, sdstep, 0)

    rb = (cid * NS + sid) * RPW

    pltpu.sync_copy(loc, shared_d.at[idxr], add=True)
    plsc.subcore_barrier()

    pltpu.sync_copy(shared_d, dtmp)

    def rstep(r, c):
        rinv[pl.ds(r * L, L)] = 1.0 / dtmp[r]
        return c

    lax.fori_loop(0, NROWS, rstep, 0)
    pltpu.async_copy(z_h, loc, sem).wait()

    wbase = cid * EP

    def wstep(i, c):
        for u in range(UNROLL):
            o = wbase + (i * UNROLL + u) * L
            sv = src_v[pl.ds(o, L)]
            dv = dst_v[pl.ds(o, L)]
            a1 = plsc.load_gather(asrc_v, [sv])
            a2 = plsc.load_gather(adst_v, [dv])
            ex = jnp.exp(a1 + a2 - m)
            ri = plsc.load_gather(rinv, [dv])
            w = ex * ri
            rv = lax.shift_right_logical(sv, 4)
            cv = lax.bitwise_and(sv, 15)
            plsc.addupdate_scatter(loc, [rv, cv], w)
        return c

    lax.fori_loop(0, EITER // UNROLL, wstep, 0)

    def swstep(rr, c):
        r = rb + rr

        @pl.when(r < N // L)
        def _():
            ex = jnp.exp(asrc_v[pl.ds(r * L, L)]
                         + adst_v[pl.ds(r * L, L)] - m)
            loc[r] = loc[r] + ex * rinv[pl.ds(r * L, L)]

        return c

    lax.fori_loop(0, RPW, swstep, 0)
    _combine_and_emit(loc, shared_c, idxr, chunk, cpart_h, cid, sid)


def _sc_edges(ei, asrc, adst, m16, zeros):
    mesh = plsc.VectorSubcoreMesh(core_axis_name="c", subcore_axis_name="s")
    scratch = [
        pltpu.VMEM((NPAD,), _f32),
        pltpu.VMEM((NPAD,), _f32),
        pltpu.VMEM((EPC,), _i32),
        pltpu.VMEM((EPC,), _i32),
        pltpu.VMEM((L,), _f32),
        pltpu.VMEM((NROWS, L), _f32),
        pltpu.VMEM((NROWS,), _i32),
        pltpu.VMEM((RPT, L), _f32),
        pltpu.SemaphoreType.DMA,
        pltpu.VMEM((NROWS, L), _f32),
        pltpu.VMEM((NPAD,), _f32),
        pltpu.VMEM_SHARED((NROWS, L), _f32),
        pltpu.VMEM_SHARED((NROWS, L), _f32),
    ]
    return functools.partial(
        pl.kernel,
        out_type=jax.ShapeDtypeStruct((NC, NROWS, L), _f32),
        mesh=mesh,
        scratch_types=scratch,
        compiler_params=pltpu.CompilerParams(needs_layout_passes=False,
                                             use_tc_tiling_on_sc=False),
    )(_edge_fused_body)(ei, asrc, adst, m16, zeros)


def _out_body(c_ref, h_ref, g_ref, wo1_ref, wo2_ref, bo_ref, o_ref, acc):
    i = pl.program_id(0)

    @pl.when(i == 0)
    def _():
        acc[...] = jnp.zeros_like(acc)

    R = h_ref.shape[0]
    c = c_ref[:, pl.ds(i * R, R)]
    acc[...] += lax.dot_general(c, h_ref[...], (((1,), (0,)), ((), ())),
                                preferred_element_type=_f32)

    @pl.when(i == pl.num_programs(0) - 1)
    def _():
        pooled = (acc[0:1, :] + acc[1:2, :]) * (1.0 / N)
        o_ref[...] = (pooled @ wo1_ref[...] + g_ref[...] @ wo2_ref[...]
                      + bo_ref[...])


def _reduce_out(c2, hidden, g_2d, wo1, wo2, bo_2d):
    R = 1024
    grid = NPAD // R
    return pl.pallas_call(
        _out_body,
        grid=(grid,),
        in_specs=[
            pl.BlockSpec((2, NPAD), lambda i: (0, 0)),
            pl.BlockSpec((R, D), lambda i: (i, 0)),
            pl.BlockSpec((1, G), lambda i: (0, 0)),
            pl.BlockSpec((D, D), lambda i: (0, 0)),
            pl.BlockSpec((G, D), lambda i: (0, 0)),
            pl.BlockSpec((1, D), lambda i: (0, 0)),
        ],
        out_specs=pl.BlockSpec((1, D), lambda i: (0, 0)),
        out_shape=jax.ShapeDtypeStruct((1, D), _f32),
        scratch_shapes=[pltpu.VMEM((2, D), _f32)],
    )(c2, hidden, g_2d, wo1, wo2, bo_2d)


def kernel(node_features, edge_index, global_features, W1, b1, Wa, ba, Wo,
           bo):
    wa8 = jnp.zeros((D, 8), _f32)
    wa8 = wa8.at[:, 0].set(Wa[:D, 0]).at[:, 1].set(Wa[D:, 0])
    hidden, asrc, adst, mmax = _encode(node_features, W1, b1.reshape(1, D),
                                       wa8)
    m16 = jnp.full((L,), mmax[0, 0] + mmax[1, 0], _f32)
    zeros = jnp.zeros((NROWS, L), _f32)

    cpart = _sc_edges(edge_index, asrc, adst, m16, zeros)

    out = _reduce_out(cpart.reshape(NC, NPAD), hidden,
                      global_features.reshape(1, G), Wo[:D], Wo[D:],
                      bo.reshape(1, D))
    return out.reshape(D)


# speedup vs baseline: 1.1381x; 1.1381x over previous
import functools
import jax
import jax.numpy as jnp
from jax import lax
from jax.experimental import pallas as pl
from jax.experimental.pallas import tpu as pltpu
from jax.experimental.pallas import tpu_sc as plsc

N = 10000
E = 320000
D = 128
G = 16

NC = 2
NS = 16
L = 16
NW = NC * NS

NPAD = 10240
NROWS = NPAD // L
RPT = NROWS // NS
RPW = NROWS // NW
EP = E // NW
EITER = EP // L
UNROLL = 25

_f32 = jnp.float32
_i32 = jnp.int32


def _enc_body(x_ref, w1_ref, b1_ref, wa_ref, h_ref, as_ref, ad_ref, m_ref):
    i = pl.program_id(0)
    R = x_ref.shape[0]
    h = x_ref[...] @ w1_ref[...] + b1_ref[...]
    h = jnp.where(h > 0, h, jnp.exp(jnp.minimum(h, 0.0)) - 1.0)
    rows = lax.broadcasted_iota(_i32, (R, 1), 0) + i * R
    h = jnp.where(rows < N, h, 0.0)
    h_ref[...] = h
    a = lax.dot_general(wa_ref[...], h, (((0,), (1,)), ((), ())),
                        preferred_element_type=_f32)
    as_ref[pl.ds(i * R, R)] = a[0]
    ad_ref[pl.ds(i * R, R)] = a[1]
    bm = jnp.max(a, axis=1, keepdims=True) + jnp.zeros((8, 128), _f32)

    @pl.when(i == 0)
    def _():
        m_ref[...] = bm

    @pl.when(i > 0)
    def _():
        m_ref[...] = jnp.maximum(m_ref[...], bm)


def _encode(x, w1, b1_2d, wa8):
    R = 1024
    grid = NPAD // R
    return pl.pallas_call(
        _enc_body,
        grid=(grid,),
        in_specs=[
            pl.BlockSpec((R, D), lambda i: (i, 0)),
            pl.BlockSpec((D, D), lambda i: (0, 0)),
            pl.BlockSpec((1, D), lambda i: (0, 0)),
            pl.BlockSpec((D, 8), lambda i: (0, 0)),
        ],
        out_specs=[
            pl.BlockSpec((R, D), lambda i: (i, 0)),
            pl.BlockSpec((NPAD,), lambda i: (0,)),
            pl.BlockSpec((NPAD,), lambda i: (0,)),
            pl.BlockSpec((8, 128), lambda i: (0, 0)),
        ],
        out_shape=[
            jax.ShapeDtypeStruct((NPAD, D), _f32),
            jax.ShapeDtypeStruct((NPAD,), _f32),
            jax.ShapeDtypeStruct((NPAD,), _f32),
            jax.ShapeDtypeStruct((8, 128), _f32),
        ],
    )(x, w1, b1_2d, wa8)


def _row_iota(ref):
    it = lax.iota(_i32, L)

    def body(r, c):
        ref[pl.ds(r * L, L)] = it + r * L
        return c

    lax.fori_loop(0, NROWS // L, body, 0)


def _combine_and_emit(loc, shared, idxr, chunk, out_h, cid, sid):
    pltpu.sync_copy(loc, shared.at[idxr], add=True)
    plsc.subcore_barrier()
    pltpu.sync_copy(shared.at[pl.ds(sid * RPT, RPT)], chunk)
    pltpu.sync_copy(chunk, out_h.at[cid, pl.ds(sid * RPT, RPT)])


SLL = RPW * L


def _denom_body(ei_h, asrc_h, adst_h, m_h, z_h, dpart_h, ex_h, asrc_v,
                adst_v, src_v, dst_v, m_v, loc, idxr, chunk, sem, ex_v,
                shared):
    cid = lax.axis_index("c")
    sid = lax.axis_index("s")
    wid = sid * NC + cid
    base = wid * EP
    cps = [
        pltpu.async_copy(asrc_h, asrc_v, sem),
        pltpu.async_copy(adst_h, adst_v, sem),
        pltpu.async_copy(ei_h.at[0, pl.ds(base, EP)], src_v, sem),
        pltpu.async_copy(ei_h.at[1, pl.ds(base, EP)], dst_v, sem),
        pltpu.async_copy(m_h, m_v, sem),
        pltpu.async_copy(z_h, loc, sem),
    ]

    @pl.when(sid == 0)
    def _():
        pltpu.async_copy(z_h, shared, sem).wait()

    _row_iota(idxr)
    for cp in cps:
        cp.wait()
    plsc.subcore_barrier()
    m = m_v[...]

    def estep(i, c):
        for u in range(UNROLL):
            o = (i * UNROLL + u) * L
            sv = src_v[pl.ds(o, L)]
            dv = dst_v[pl.ds(o, L)]
            a1 = plsc.load_gather(asrc_v, [sv])
            a2 = plsc.load_gather(adst_v, [dv])
            ex = jnp.exp(a1 + a2 - m)
            ex_v[pl.ds(o, L)] = ex
            rv = lax.shift_right_logical(dv, 4)
            cv = lax.bitwise_and(dv, 15)
            plsc.addupdate_scatter(loc, [rv, cv], ex)
        return c

    lax.fori_loop(0, EITER // UNROLL, estep, 0)
    exout = pltpu.async_copy(ex_v, ex_h.at[wid], sem)

    rb = (cid * NS + sid) * RPW

    def sstep(rr, c):
        r = rb + rr

        @pl.when(r < N // L)
        def _():
            ex = jnp.exp(asrc_v[pl.ds(r * L, L)]
                         + adst_v[pl.ds(r * L, L)] - m)
            loc[r] = loc[r] + ex

        return c

    lax.fori_loop(0, RPW, sstep, 0)
    exout.wait()
    _combine_and_emit(loc, shared, idxr, chunk, dpart_h, cid, sid)


def _weight_body(ei_h, asrc_h, adst_h, m_h, z_h, dpart_h, ex_h, cpart_h,
                 asl_v, adl_v, src_v, dst_v, m_v, loc, idxr, chunk, sem,
                 ex_v, d0, d1, rinv, shared):
    cid = lax.axis_index("c")
    sid = lax.axis_index("s")
    wid = sid * NC + cid
    base = wid * EP
    rb = (cid * NS + sid) * RPW
    cps = [
        pltpu.async_copy(asrc_h.at[pl.ds(rb * L, SLL)], asl_v, sem),
        pltpu.async_copy(adst_h.at[pl.ds(rb * L, SLL)], adl_v, sem),
        pltpu.async_copy(ei_h.at[0, pl.ds(base, EP)], src_v, sem),
        pltpu.async_copy(ei_h.at[1, pl.ds(base, EP)], dst_v, sem),
        pltpu.async_copy(m_h, m_v, sem),
        pltpu.async_copy(z_h, loc, sem),
        pltpu.async_copy(ex_h.at[wid], ex_v, sem),
        pltpu.async_copy(dpart_h.at[0], d0, sem),
        pltpu.async_copy(dpart_h.at[1], d1, sem),
    ]

    @pl.when(sid == 0)
    def _():
        pltpu.async_copy(z_h, shared, sem).wait()

    _row_iota(idxr)
    for cp in cps:
        cp.wait()
    plsc.subcore_barrier()
    m = m_v[...]

    def rstep(r, c):
        rinv[pl.ds(r * L, L)] = 1.0 / (d0[r] + d1[r])
        return c

    lax.fori_loop(0, NROWS, rstep, 0)

    def estep(i, c):
        for u in range(UNROLL):
            o = (i * UNROLL + u) * L
            sv = src_v[pl.ds(o, L)]
            dv = dst_v[pl.ds(o, L)]
            ex = ex_v[pl.ds(o, L)]
            ri = plsc.load_gather(rinv, [dv])
            w = ex * ri
            rv = lax.shift_right_logical(sv, 4)
            cv = lax.bitwise_and(sv, 15)
            plsc.addupdate_scatter(loc, [rv, cv], w)
        return c

    lax.fori_loop(0, EITER // UNROLL, estep, 0)

    def sstep(rr, c):
        r = rb + rr

        @pl.when(r < N // L)
        def _():
            ex = jnp.exp(asl_v[pl.ds(rr * L, L)]
                         + adl_v[pl.ds(rr * L, L)] - m)
            loc[r] = loc[r] + ex * rinv[pl.ds(r * L, L)]

        return c

    lax.fori_loop(0, RPW, sstep, 0)
    _combine_and_emit(loc, shared, idxr, chunk, cpart_h, cid, sid)


def _sc_mesh_kernel(body, out_type, extra_scratch):
    mesh = plsc.VectorSubcoreMesh(core_axis_name="c", subcore_axis_name="s")
    scratch = [
        pltpu.VMEM((EP,), _i32),
        pltpu.VMEM((EP,), _i32),
        pltpu.VMEM((L,), _f32),
        pltpu.VMEM((NROWS, L), _f32),
        pltpu.VMEM((NROWS,), _i32),
        pltpu.VMEM((RPT, L), _f32),
        pltpu.SemaphoreType.DMA,
        pltpu.VMEM((EP,), _f32),
    ] + extra_scratch + [
        pltpu.VMEM_SHARED((NROWS, L), _f32),
    ]
    return functools.partial(
        pl.kernel,
        out_type=out_type,
        mesh=mesh,
        scratch_types=scratch,
        compiler_params=pltpu.CompilerParams(needs_layout_passes=False,
                                             use_tc_tiling_on_sc=False),
    )(body)


def _sc_denom(ei, asrc, adst, m16, zeros):
    out_type = [
        jax.ShapeDtypeStruct((NC, NROWS, L), _f32),
        jax.ShapeDtypeStruct((NW, EP), _f32),
    ]
    extra = [
        pltpu.VMEM((NPAD,), _f32),
        pltpu.VMEM((NPAD,), _f32),
    ]
    def body(ei_h, asrc_h, adst_h, m_h, z_h, dpart_h, ex_h, src_v, dst_v,
             m_v, loc, idxr, chunk, sem, ex_v, asrc_v, adst_v, shared):
        _denom_body(ei_h, asrc_h, adst_h, m_h, z_h, dpart_h, ex_h, asrc_v,
                    adst_v, src_v, dst_v, m_v, loc, idxr, chunk, sem, ex_v,
                    shared)

    return _sc_mesh_kernel(body, out_type, extra)(ei, asrc, adst, m16,
                                                  zeros)


def _sc_weight(ei, asrc, adst, m16, zeros, dpart, exall):
    out_type = jax.ShapeDtypeStruct((NC, NROWS, L), _f32)
    extra = [
        pltpu.VMEM((SLL,), _f32),
        pltpu.VMEM((SLL,), _f32),
        pltpu.VMEM((NROWS, L), _f32),
        pltpu.VMEM((NROWS, L), _f32),
        pltpu.VMEM((NPAD,), _f32),
    ]

    def body(ei_h, asrc_h, adst_h, m_h, z_h, dpart_h, ex_h, cpart_h, src_v,
             dst_v, m_v, loc, idxr, chunk, sem, ex_v, asl_v, adl_v, d0, d1,
             rinv, shared):
        _weight_body(ei_h, asrc_h, adst_h, m_h, z_h, dpart_h, ex_h, cpart_h,
                     asl_v, adl_v, src_v, dst_v, m_v, loc, idxr, chunk, sem,
                     ex_v, d0, d1, rinv, shared)

    return _sc_mesh_kernel(body, out_type, extra)(ei, asrc, adst, m16,
                                                  zeros, dpart, exall)


def _out_body(c_ref, h_ref, g_ref, wo1_ref, wo2_ref, bo_ref, o_ref, acc):
    i = pl.program_id(0)

    @pl.when(i == 0)
    def _():
        acc[...] = jnp.zeros_like(acc)

    R = h_ref.shape[0]
    c = c_ref[:, pl.ds(i * R, R)]
    acc[...] += lax.dot_general(c, h_ref[...], (((1,), (0,)), ((), ())),
                                preferred_element_type=_f32)

    @pl.when(i == pl.num_programs(0) - 1)
    def _():
        pooled = (acc[0:1, :] + acc[1:2, :]) * (1.0 / N)
        o_ref[...] = (pooled @ wo1_ref[...] + g_ref[...] @ wo2_ref[...]
                      + bo_ref[...])


def _reduce_out(c2, hidden, g_2d, wo1, wo2, bo_2d):
    R = 1024
    grid = NPAD // R
    return pl.pallas_call(
        _out_body,
        grid=(grid,),
        in_specs=[
            pl.BlockSpec((2, NPAD), lambda i: (0, 0)),
            pl.BlockSpec((R, D), lambda i: (i, 0)),
            pl.BlockSpec((1, G), lambda i: (0, 0)),
            pl.BlockSpec((D, D), lambda i: (0, 0)),
            pl.BlockSpec((G, D), lambda i: (0, 0)),
            pl.BlockSpec((1, D), lambda i: (0, 0)),
        ],
        out_specs=pl.BlockSpec((1, D), lambda i: (0, 0)),
        out_shape=jax.ShapeDtypeStruct((1, D), _f32),
        scratch_shapes=[pltpu.VMEM((2, D), _f32)],
    )(c2, hidden, g_2d, wo1, wo2, bo_2d)


def kernel(node_features, edge_index, global_features, W1, b1, Wa, ba, Wo,
           bo):
    wa8 = jnp.zeros((D, 8), _f32)
    wa8 = wa8.at[:, 0].set(Wa[:D, 0]).at[:, 1].set(Wa[D:, 0])
    hidden, asrc, adst, mmax = _encode(node_features, W1, b1.reshape(1, D),
                                       wa8)
    m16 = jnp.full((L,), mmax[0, 0] + mmax[1, 0], _f32)
    zeros = jnp.zeros((NROWS, L), _f32)

    dpart, exall = _sc_denom(edge_index, asrc, adst, m16, zeros)
    cpart = _sc_weight(edge_index, asrc, adst, m16, zeros, dpart, exall)

    out = _reduce_out(cpart.reshape(NC, NPAD), hidden,
                      global_features.reshape(1, G), Wo[:D], Wo[D:],
                      bo.reshape(1, D))
    return out.reshape(D)


# speedup vs baseline: 1.1441x; 1.0053x over previous
import functools
import jax
import jax.numpy as jnp
from jax import lax
from jax.experimental import pallas as pl
from jax.experimental.pallas import tpu as pltpu
from jax.experimental.pallas import tpu_sc as plsc

N = 10000
E = 320000
D = 128
G = 16

NC = 2
NS = 16
L = 16
NW = NC * NS

NPAD = 10240
NROWS = NPAD // L
RPT = NROWS // NS
RPW = NROWS // NW
EP = E // NW
EITER = EP // L
UNROLL = 5

_f32 = jnp.float32
_i32 = jnp.int32


def _enc_body(x_ref, w1_ref, b1_ref, wa_ref, h_ref, as_ref, ad_ref, m_ref):
    i = pl.program_id(0)
    R = x_ref.shape[0]
    h = x_ref[...] @ w1_ref[...] + b1_ref[...]
    h = jnp.where(h > 0, h, jnp.exp(jnp.minimum(h, 0.0)) - 1.0)
    rows = lax.broadcasted_iota(_i32, (R, 1), 0) + i * R
    h = jnp.where(rows < N, h, 0.0)
    h_ref[...] = h
    a = lax.dot_general(wa_ref[...], h, (((0,), (1,)), ((), ())),
                        preferred_element_type=_f32)
    as_ref[pl.ds(i * R, R)] = a[0]
    ad_ref[pl.ds(i * R, R)] = a[1]
    bm = jnp.max(a, axis=1, keepdims=True) + jnp.zeros((8, 128), _f32)

    @pl.when(i == 0)
    def _():
        m_ref[...] = bm

    @pl.when(i > 0)
    def _():
        m_ref[...] = jnp.maximum(m_ref[...], bm)


def _encode(x, w1, b1_2d, wa8):
    R = 1024
    grid = NPAD // R
    return pl.pallas_call(
        _enc_body,
        grid=(grid,),
        in_specs=[
            pl.BlockSpec((R, D), lambda i: (i, 0)),
            pl.BlockSpec((D, D), lambda i: (0, 0)),
            pl.BlockSpec((1, D), lambda i: (0, 0)),
            pl.BlockSpec((D, 8), lambda i: (0, 0)),
        ],
        out_specs=[
            pl.BlockSpec((R, D), lambda i: (i, 0)),
            pl.BlockSpec((NPAD,), lambda i: (0,)),
            pl.BlockSpec((NPAD,), lambda i: (0,)),
            pl.BlockSpec((8, 128), lambda i: (0, 0)),
        ],
        out_shape=[
            jax.ShapeDtypeStruct((NPAD, D), _f32),
            jax.ShapeDtypeStruct((NPAD,), _f32),
            jax.ShapeDtypeStruct((NPAD,), _f32),
            jax.ShapeDtypeStruct((8, 128), _f32),
        ],
    )(x, w1, b1_2d, wa8)


def _row_iota(ref):
    it = lax.iota(_i32, L)

    def body(r, c):
        ref[pl.ds(r * L, L)] = it + r * L
        return c

    lax.fori_loop(0, NROWS // L, body, 0)


def _combine_and_emit(loc, shared, idxr, chunk, out_h, cid, sid):
    pltpu.sync_copy(loc, shared.at[idxr], add=True)
    plsc.subcore_barrier()
    pltpu.sync_copy(shared.at[pl.ds(sid * RPT, RPT)], chunk)
    pltpu.sync_copy(chunk, out_h.at[cid, pl.ds(sid * RPT, RPT)])


SLL = RPW * L


def _denom_body(ei_h, asrc_h, adst_h, m_h, z_h, dpart_h, ex_h, asrc_v,
                adst_v, src_v, dst_v, m_v, loc, idxr, chunk, sem, ex_v,
                shared):
    cid = lax.axis_index("c")
    sid = lax.axis_index("s")
    wid = sid * NC + cid
    base = wid * EP
    cps = [
        pltpu.async_copy(asrc_h, asrc_v, sem),
        pltpu.async_copy(adst_h, adst_v, sem),
        pltpu.async_copy(ei_h.at[0, pl.ds(base, EP)], src_v, sem),
        pltpu.async_copy(ei_h.at[1, pl.ds(base, EP)], dst_v, sem),
        pltpu.async_copy(m_h, m_v, sem),
        pltpu.async_copy(z_h, loc, sem),
    ]

    @pl.when(sid == 0)
    def _():
        pltpu.async_copy(z_h, shared, sem).wait()

    _row_iota(idxr)
    for cp in cps:
        cp.wait()
    plsc.subcore_barrier()
    m = m_v[...]

    def estep(i, c):
        for u in range(UNROLL):
            o = (i * UNROLL + u) * L
            sv = src_v[pl.ds(o, L)]
            dv = dst_v[pl.ds(o, L)]
            a1 = plsc.load_gather(asrc_v, [sv])
            a2 = plsc.load_gather(adst_v, [dv])
            ex = jnp.exp(a1 + a2 - m)
            ex_v[pl.ds(o, L)] = ex
            rv = lax.shift_right_logical(dv, 4)
            cv = lax.bitwise_and(dv, 15)
            plsc.addupdate_scatter(loc, [rv, cv], ex)
        return c

    lax.fori_loop(0, EITER // UNROLL, estep, 0)
    exout = pltpu.async_copy(ex_v, ex_h.at[wid], sem)

    rb = (cid * NS + sid) * RPW

    def sstep(rr, c):
        r = rb + rr

        @pl.when(r < N // L)
        def _():
            ex = jnp.exp(asrc_v[pl.ds(r * L, L)]
                         + adst_v[pl.ds(r * L, L)] - m)
            loc[r] = loc[r] + ex

        return c

    lax.fori_loop(0, RPW, sstep, 0)
    exout.wait()
    _combine_and_emit(loc, shared, idxr, chunk, dpart_h, cid, sid)


def _weight_body(ei_h, asrc_h, adst_h, m_h, z_h, dpart_h, ex_h, cpart_h,
                 asl_v, adl_v, src_v, dst_v, m_v, loc, idxr, chunk, sem,
                 ex_v, d0, d1, rinv, shared):
    cid = lax.axis_index("c")
    sid = lax.axis_index("s")
    wid = sid * NC + cid
    base = wid * EP
    rb = (cid * NS + sid) * RPW
    cps = [
        pltpu.async_copy(asrc_h.at[pl.ds(rb * L, SLL)], asl_v, sem),
        pltpu.async_copy(adst_h.at[pl.ds(rb * L, SLL)], adl_v, sem),
        pltpu.async_copy(ei_h.at[0, pl.ds(base, EP)], src_v, sem),
        pltpu.async_copy(ei_h.at[1, pl.ds(base, EP)], dst_v, sem),
        pltpu.async_copy(m_h, m_v, sem),
        pltpu.async_copy(z_h, loc, sem),
        pltpu.async_copy(ex_h.at[wid], ex_v, sem),
        pltpu.async_copy(dpart_h.at[0], d0, sem),
        pltpu.async_copy(dpart_h.at[1], d1, sem),
    ]

    @pl.when(sid == 0)
    def _():
        pltpu.async_copy(z_h, shared, sem).wait()

    _row_iota(idxr)
    for cp in cps:
        cp.wait()
    plsc.subcore_barrier()
    m = m_v[...]

    def rstep(r, c):
        rinv[pl.ds(r * L, L)] = 1.0 / (d0[r] + d1[r])
        return c

    lax.fori_loop(0, NROWS, rstep, 0)

    def estep(i, c):
        for u in range(UNROLL):
            o = (i * UNROLL + u) * L
            sv = src_v[pl.ds(o, L)]
            dv = dst_v[pl.ds(o, L)]
            ex = ex_v[pl.ds(o, L)]
            ri = plsc.load_gather(rinv, [dv])
            w = ex * ri
            rv = lax.shift_right_logical(sv, 4)
            cv = lax.bitwise_and(sv, 15)
            plsc.addupdate_scatter(loc, [rv, cv], w)
        return c

    lax.fori_loop(0, EITER // UNROLL, estep, 0)

    def sstep(rr, c):
        r = rb + rr

        @pl.when(r < N // L)
        def _():
            ex = jnp.exp(asl_v[pl.ds(rr * L, L)]
                         + adl_v[pl.ds(rr * L, L)] - m)
            loc[r] = loc[r] + ex * rinv[pl.ds(r * L, L)]

        return c

    lax.fori_loop(0, RPW, sstep, 0)
    _combine_and_emit(loc, shared, idxr, chunk, cpart_h, cid, sid)


def _sc_mesh_kernel(body, out_type, extra_scratch):
    mesh = plsc.VectorSubcoreMesh(core_axis_name="c", subcore_axis_name="s")
    scratch = [
        pltpu.VMEM((EP,), _i32),
        pltpu.VMEM((EP,), _i32),
        pltpu.VMEM((L,), _f32),
        pltpu.VMEM((NROWS, L), _f32),
        pltpu.VMEM((NROWS,), _i32),
        pltpu.VMEM((RPT, L), _f32),
        pltpu.SemaphoreType.DMA,
        pltpu.VMEM((EP,), _f32),
    ] + extra_scratch + [
        pltpu.VMEM_SHARED((NROWS, L), _f32),
    ]
    return functools.partial(
        pl.kernel,
        out_type=out_type,
        mesh=mesh,
        scratch_types=scratch,
        compiler_params=pltpu.CompilerParams(needs_layout_passes=False,
                                             use_tc_tiling_on_sc=False),
    )(body)


def _sc_denom(ei, asrc, adst, m16, zeros):
    out_type = [
        jax.ShapeDtypeStruct((NC, NROWS, L), _f32),
        jax.ShapeDtypeStruct((NW, EP), _f32),
    ]
    extra = [
        pltpu.VMEM((NPAD,), _f32),
        pltpu.VMEM((NPAD,), _f32),
    ]
    def body(ei_h, asrc_h, adst_h, m_h, z_h, dpart_h, ex_h, src_v, dst_v,
             m_v, loc, idxr, chunk, sem, ex_v, asrc_v, adst_v, shared):
        _denom_body(ei_h, asrc_h, adst_h, m_h, z_h, dpart_h, ex_h, asrc_v,
                    adst_v, src_v, dst_v, m_v, loc, idxr, chunk, sem, ex_v,
                    shared)

    return _sc_mesh_kernel(body, out_type, extra)(ei, asrc, adst, m16,
                                                  zeros)


def _sc_weight(ei, asrc, adst, m16, zeros, dpart, exall):
    out_type = jax.ShapeDtypeStruct((NC, NROWS, L), _f32)
    extra = [
        pltpu.VMEM((SLL,), _f32),
        pltpu.VMEM((SLL,), _f32),
        pltpu.VMEM((NROWS, L), _f32),
        pltpu.VMEM((NROWS, L), _f32),
        pltpu.VMEM((NPAD,), _f32),
    ]

    def body(ei_h, asrc_h, adst_h, m_h, z_h, dpart_h, ex_h, cpart_h, src_v,
             dst_v, m_v, loc, idxr, chunk, sem, ex_v, asl_v, adl_v, d0, d1,
             rinv, shared):
        _weight_body(ei_h, asrc_h, adst_h, m_h, z_h, dpart_h, ex_h, cpart_h,
                     asl_v, adl_v, src_v, dst_v, m_v, loc, idxr, chunk, sem,
                     ex_v, d0, d1, rinv, shared)

    return _sc_mesh_kernel(body, out_type, extra)(ei, asrc, adst, m16,
                                                  zeros, dpart, exall)


def _out_body(c_ref, h_ref, g_ref, wo1_ref, wo2_ref, bo_ref, o_ref, acc):
    i = pl.program_id(0)

    @pl.when(i == 0)
    def _():
        acc[...] = jnp.zeros_like(acc)

    R = h_ref.shape[0]
    c = c_ref[:, pl.ds(i * R, R)]
    acc[...] += lax.dot_general(c, h_ref[...], (((1,), (0,)), ((), ())),
                                preferred_element_type=_f32)

    @pl.when(i == pl.num_programs(0) - 1)
    def _():
        pooled = (acc[0:1, :] + acc[1:2, :]) * (1.0 / N)
        o_ref[...] = (pooled @ wo1_ref[...] + g_ref[...] @ wo2_ref[...]
                      + bo_ref[...])


def _reduce_out(c2, hidden, g_2d, wo1, wo2, bo_2d):
    R = 1024
    grid = NPAD // R
    return pl.pallas_call(
        _out_body,
        grid=(grid,),
        in_specs=[
            pl.BlockSpec((2, NPAD), lambda i: (0, 0)),
            pl.BlockSpec((R, D), lambda i: (i, 0)),
            pl.BlockSpec((1, G), lambda i: (0, 0)),
            pl.BlockSpec((D, D), lambda i: (0, 0)),
            pl.BlockSpec((G, D), lambda i: (0, 0)),
            pl.BlockSpec((1, D), lambda i: (0, 0)),
        ],
        out_specs=pl.BlockSpec((1, D), lambda i: (0, 0)),
        out_shape=jax.ShapeDtypeStruct((1, D), _f32),
        scratch_shapes=[pltpu.VMEM((2, D), _f32)],
    )(c2, hidden, g_2d, wo1, wo2, bo_2d)


def kernel(node_features, edge_index, global_features, W1, b1, Wa, ba, Wo,
           bo):
    wa8 = jnp.zeros((D, 8), _f32)
    wa8 = wa8.at[:, 0].set(Wa[:D, 0]).at[:, 1].set(Wa[D:, 0])
    hidden, asrc, adst, mmax = _encode(node_features, W1, b1.reshape(1, D),
                                       wa8)
    m16 = jnp.full((L,), mmax[0, 0] + mmax[1, 0], _f32)
    zeros = jnp.zeros((NROWS, L), _f32)

    dpart, exall = _sc_denom(edge_index, asrc, adst, m16, zeros)
    cpart = _sc_weight(edge_index, asrc, adst, m16, zeros, dpart, exall)

    out = _reduce_out(cpart.reshape(NC, NPAD), hidden,
                      global_features.reshape(1, G), Wo[:D], Wo[D:],
                      bo.reshape(1, D))
    return out.reshape(D)
